# baseline (speedup 1.0000x reference)
import functools
import jax
import jax.numpy as jnp
from jax import lax
from jax.experimental import pallas as pl
from jax.experimental.pallas import tpu as pltpu
from jax.experimental.pallas import tpu_sc as plsc

NC = 2
NS = 16
NW = NC * NS
BSTAGE = 4
SPLITS = (0, 104, 200)

BB = 128
L = 16
RPAD = 65


def _gather_kernel(B_, S, V, D, bat_per_w, n_stage):
    mesh = plsc.VectorSubcoreMesh(core_axis_name="c", subcore_axis_name="s")

    @functools.partial(
        pl.kernel,
        out_type=jax.ShapeDtypeStruct((B_, S, D), jnp.float32),
        mesh=mesh,
        scratch_types=[
            pltpu.VMEM((2, BSTAGE, S), jnp.int32),
            pltpu.VMEM((2, BSTAGE, S, D), jnp.float32),
            pltpu.SemaphoreType.DMA,
            pltpu.SemaphoreType.DMA,
            pltpu.SemaphoreType.DMA,
            pltpu.SemaphoreType.DMA,
            pltpu.SemaphoreType.DMA,
            pltpu.SemaphoreType.DMA,
        ],
        compiler_params=pltpu.CompilerParams(use_tc_tiling_on_sc=False),
    )
    def k(idx_hbm, table_hbm, out_hbm, idx_v, rows_v, g0, g1, s0, s1, i0, i1):
        gsem = (g0, g1)
        ssem = (s0, s1)
        isem = (i0, i1)
        wid = lax.axis_index("s") * NC + lax.axis_index("c")
        bat0 = wid * bat_per_w

        def idx_copy(g, slot):
            return pltpu.make_async_copy(
                idx_hbm.at[pl.ds(bat0 + g * BSTAGE, BSTAGE)],
                idx_v.at[slot],
                isem[slot],
            )

        def gather_copies(slot):
            out = []
            for i in range(BSTAGE):
                for c in range(len(SPLITS) - 1):
                    lo, hi = SPLITS[c], SPLITS[c + 1]
                    out.append(
                        pltpu.make_async_copy(
                            table_hbm.at[idx_v.at[slot, i, pl.ds(lo, hi - lo)]],
                            rows_v.at[slot, i, pl.ds(lo, hi - lo)],
                            gsem[slot],
                        )
                    )
            return out

        def store_copy(g, slot):
            return pltpu.make_async_copy(
                rows_v.at[slot],
                out_hbm.at[pl.ds(bat0 + g * BSTAGE, BSTAGE)],
                ssem[slot],
            )

        def stage(g, b, first=False, prefetch=True):
            nb = 1 - b
            for d in gather_copies(b):
                d.wait()
            store_copy(g, b).start()
            idx_copy(g + 1, nb).wait()
            if not first:
                store_copy(g - 1, nb).wait()
            for d in gather_copies(nb):
                d.start()
            if prefetch:
                idx_copy(g + 2, b).start()

        idx_copy(0, 0).start()
        idx_copy(0, 0).wait()
        for d in gather_copies(0):
            d.start()
        idx_copy(1, 1).start()

        stage(0, 0, first=True)
        stage(1, 1)

        def body(h, carry):
            for b in range(2):
                stage(2 * h + b, b)
            return carry

        lax.fori_loop(1, n_stage // 2 - 1, body, 0)

        stage(n_stage - 2, 0, prefetch=False)
        for d in gather_copies(1):
            d.wait()
        store_copy(n_stage - 1, 1).start()

        store_copy(n_stage - 2, 0).wait()
        store_copy(n_stage - 1, 1).wait()

    return k


def _format_kernel(B_, S, D, blk_per_w):
    mesh = plsc.VectorSubcoreMesh(core_axis_name="c", subcore_axis_name="s")
    DB = D // 8

    @functools.partial(
        pl.kernel,
        out_type=jax.ShapeDtypeStruct((S, DB, B_ // BB, 8, BB), jnp.float32),
        mesh=mesh,
        scratch_types=[
            pltpu.VMEM((2, BB, RPAD), jnp.float32),
            pltpu.VMEM((2, DB, 8, BB), jnp.float32),
            pltpu.SemaphoreType.DMA,
            pltpu.SemaphoreType.DMA,
            pltpu.SemaphoreType.DMA,
            pltpu.SemaphoreType.DMA,
        ],
        compiler_params=pltpu.CompilerParams(
            use_tc_tiling_on_sc=False, needs_layout_passes=False
        ),
    )
    def k(x_hbm, out_hbm, rows_v, tile_v, g0, g1, s0, s1):
        gsem = (g0, g1)
        ssem = (s0, s1)
        wid = lax.axis_index("s") * NC + lax.axis_index("c")
        bblk0 = wid * blk_per_w

        lane = lax.iota(jnp.int32, L)
        row_base = [(lane + h * L) * RPAD for h in range(BB // L)]

        def load_copy(s, slot, bblk):
            return pltpu.make_async_copy(
                x_hbm.at[pl.ds((bblk0 + bblk) * BB, BB), s],
                rows_v.at[slot, :, pl.ds(0, D)],
                gsem[slot],
            )

        def store_copy(s, slot, bblk):
            return pltpu.make_async_copy(
                tile_v.at[slot],
                out_hbm.at[s, :, bblk0 + bblk],
                ssem[slot],
            )

        def full_transpose(slot):
            @plsc.parallel_loop(0, D, step=1, unroll=32)
            def dbody(d):
                col = jnp.full((L,), 0, jnp.int32) + d
                for h in range(BB // L):
                    v = plsc.load_gather(
                        rows_v.at[slot], [lane + h * L, col]
                    )
                    tile_v[slot, d // 8, d % 8, pl.ds(h * L, L)] = v

        for bblk in range(blk_per_w):
            load_copy(0, 0, bblk).start()

            def sbody(h2, carry):
                for b in range(2):
                    s = 2 * h2 + b
                    nb = 1 - b
                    load_copy(s, b, bblk).wait()

                    @pl.when(s + 1 < S)
                    def _():
                        load_copy(s + 1, nb, bblk).start()

                    @pl.when(s >= 2)
                    def _():
                        store_copy(s - 2, b, bblk).wait()

                    full_transpose(b)
                    store_copy(s, b, bblk).start()
                return carry

            lax.fori_loop(0, S // 2, sbody, 0)

            store_copy(S - 2, 0, bblk).wait()
            store_copy(S - 1, 1, bblk).wait()

    return k


def kernel(intent_ids, table):
    B_, S = intent_ids.shape
    V, D = table.shape
    assert B_ % (NW * BB) == 0 and B_ % (NW * BSTAGE) == 0
    assert D % 8 == 0 and S % 2 == 0
    bat_per_w = B_ // NW
    n_stage = bat_per_w // BSTAGE
    blk_per_w = B_ // (NW * BB)

    x = _gather_kernel(B_, S, V, D, bat_per_w, n_stage)(intent_ids, table)
    out5 = _format_kernel(B_, S, D, blk_per_w)(x)
    return out5.transpose(2, 4, 0, 1, 3).reshape(B_, S, D)


# speedup vs baseline: 1.0215x; 1.0215x over previous
import functools
import jax
import jax.numpy as jnp
from jax import lax
from jax.experimental import pallas as pl
from jax.experimental.pallas import tpu as pltpu
from jax.experimental.pallas import tpu_sc as plsc

NC = 2
NS = 16
NW = NC * NS
BSTAGE = 4
SPLITS = (0, 104, 200)

BB = 128
L = 16
RPAD = 65


def _gather_kernel(B_, S, V, D, bat_per_w, n_stage):
    mesh = plsc.VectorSubcoreMesh(core_axis_name="c", subcore_axis_name="s")

    @functools.partial(
        pl.kernel,
        out_type=jax.ShapeDtypeStruct((B_, S, D), jnp.float32),
        mesh=mesh,
        scratch_types=[
            pltpu.VMEM((2, BSTAGE, S), jnp.int32),
            pltpu.VMEM((2, BSTAGE, S, D), jnp.float32),
            pltpu.SemaphoreType.DMA,
            pltpu.SemaphoreType.DMA,
            pltpu.SemaphoreType.DMA,
            pltpu.SemaphoreType.DMA,
            pltpu.SemaphoreType.DMA,
            pltpu.SemaphoreType.DMA,
        ],
        compiler_params=pltpu.CompilerParams(use_tc_tiling_on_sc=False),
    )
    def k(idx_hbm, table_hbm, out_hbm, idx_v, rows_v, g0, g1, s0, s1, i0, i1):
        gsem = (g0, g1)
        ssem = (s0, s1)
        isem = (i0, i1)
        wid = lax.axis_index("s") * NC + lax.axis_index("c")
        bat0 = wid * bat_per_w

        def idx_copy(g, slot):
            return pltpu.make_async_copy(
                idx_hbm.at[pl.ds(bat0 + g * BSTAGE, BSTAGE)],
                idx_v.at[slot],
                isem[slot],
            )

        def gather_copies(slot):
            out = []
            for i in range(BSTAGE):
                for c in range(len(SPLITS) - 1):
                    lo, hi = SPLITS[c], SPLITS[c + 1]
                    out.append(
                        pltpu.make_async_copy(
                            table_hbm.at[idx_v.at[slot, i, pl.ds(lo, hi - lo)]],
                            rows_v.at[slot, i, pl.ds(lo, hi - lo)],
                            gsem[slot],
                        )
                    )
            return out

        def store_copy(g, slot):
            return pltpu.make_async_copy(
                rows_v.at[slot],
                out_hbm.at[pl.ds(bat0 + g * BSTAGE, BSTAGE)],
                ssem[slot],
            )

        def stage(g, b, first=False, prefetch=True):
            nb = 1 - b
            for d in gather_copies(b):
                d.wait()
            store_copy(g, b).start()
            idx_copy(g + 1, nb).wait()
            if not first:
                store_copy(g - 1, nb).wait()
            for d in gather_copies(nb):
                d.start()
            if prefetch:
                idx_copy(g + 2, b).start()

        idx_copy(0, 0).start()
        idx_copy(0, 0).wait()
        for d in gather_copies(0):
            d.start()
        idx_copy(1, 1).start()

        stage(0, 0, first=True)
        stage(1, 1)

        def body(h, carry):
            for b in range(2):
                stage(2 * h + b, b)
            return carry

        lax.fori_loop(1, n_stage // 2 - 1, body, 0)

        stage(n_stage - 2, 0, prefetch=False)
        for d in gather_copies(1):
            d.wait()
        store_copy(n_stage - 1, 1).start()

        store_copy(n_stage - 2, 0).wait()
        store_copy(n_stage - 1, 1).wait()

    return k


def _format_kernel(B_, S, D, blk_per_w):
    mesh = plsc.VectorSubcoreMesh(core_axis_name="c", subcore_axis_name="s")
    DB = D // 8

    @functools.partial(
        pl.kernel,
        out_type=jax.ShapeDtypeStruct((S, DB, B_ // BB, 8, BB), jnp.float32),
        mesh=mesh,
        scratch_types=[
            pltpu.VMEM((2, BB, RPAD), jnp.float32),
            pltpu.VMEM((2, DB, 8, BB), jnp.float32),
            pltpu.SemaphoreType.DMA,
            pltpu.SemaphoreType.DMA,
            pltpu.SemaphoreType.DMA,
            pltpu.SemaphoreType.DMA,
        ],
        compiler_params=pltpu.CompilerParams(
            use_tc_tiling_on_sc=False, needs_layout_passes=False
        ),
    )
    def k(x_hbm, out_hbm, rows_v, tile_v, g0, g1, s0, s1):
        gsem = (g0, g1)
        ssem = (s0, s1)
        wid = lax.axis_index("s") * NC + lax.axis_index("c")
        bblk0 = wid * blk_per_w

        lane = lax.iota(jnp.int32, L)
        row_base = [(lane + h * L) * RPAD for h in range(BB // L)]

        def load_copy(s, slot, bblk):
            return pltpu.make_async_copy(
                x_hbm.at[pl.ds((bblk0 + bblk) * BB, BB), s],
                rows_v.at[slot, :, pl.ds(0, D)],
                gsem[slot],
            )

        def store_copy(s, slot, bblk):
            return pltpu.make_async_copy(
                tile_v.at[slot],
                out_hbm.at[s, :, bblk0 + bblk],
                ssem[slot],
            )

        def full_transpose(slot):
            @plsc.parallel_loop(0, D, step=1, unroll=16)
            def dbody(d):
                col = jnp.full((L,), 0, jnp.int32) + d
                for h in range(BB // L):
                    v = plsc.load_gather(
                        rows_v.at[slot], [lane + h * L, col]
                    )
                    tile_v[slot, d // 8, d % 8, pl.ds(h * L, L)] = v

        for bblk in range(blk_per_w):
            load_copy(0, 0, bblk).start()

            def sbody(h2, carry):
                for b in range(2):
                    s = 2 * h2 + b
                    nb = 1 - b
                    load_copy(s, b, bblk).wait()

                    @pl.when(s + 1 < S)
                    def _():
                        load_copy(s + 1, nb, bblk).start()

                    @pl.when(s >= 2)
                    def _():
                        store_copy(s - 2, b, bblk).wait()

                    full_transpose(b)
                    store_copy(s, b, bblk).start()
                return carry

            lax.fori_loop(0, S // 2, sbody, 0)

            store_copy(S - 2, 0, bblk).wait()
            store_copy(S - 1, 1, bblk).wait()

    return k


def kernel(intent_ids, table):
    B_, S = intent_ids.shape
    V, D = table.shape
    assert B_ % (NW * BB) == 0 and B_ % (NW * BSTAGE) == 0
    assert D % 8 == 0 and S % 2 == 0
    bat_per_w = B_ // NW
    n_stage = bat_per_w // BSTAGE
    blk_per_w = B_ // (NW * BB)

    x = _gather_kernel(B_, S, V, D, bat_per_w, n_stage)(intent_ids, table)
    out5 = _format_kernel(B_, S, D, blk_per_w)(x)
    return out5.transpose(2, 4, 0, 1, 3).reshape(B_, S, D)


# speedup vs baseline: 1.0872x; 1.0644x over previous
import functools
import jax
import jax.numpy as jnp
from jax import lax
from jax.experimental import pallas as pl
from jax.experimental.pallas import tpu as pltpu
from jax.experimental.pallas import tpu_sc as plsc

NC = 2
NS = 16
NW = NC * NS
BSTAGE = 4
SPLITS = (0, 104, 200)

BB = 128
L = 16
RPAD = 65


def _gather_kernel(B_, S, V, D, bat_per_w, n_stage):
    mesh = plsc.VectorSubcoreMesh(core_axis_name="c", subcore_axis_name="s")

    @functools.partial(
        pl.kernel,
        out_type=jax.ShapeDtypeStruct((B_, S, D), jnp.float32),
        mesh=mesh,
        scratch_types=[
            pltpu.VMEM((2, BSTAGE, S), jnp.int32),
            pltpu.VMEM((2, BSTAGE, S, D), jnp.float32),
            pltpu.SemaphoreType.DMA,
            pltpu.SemaphoreType.DMA,
            pltpu.SemaphoreType.DMA,
            pltpu.SemaphoreType.DMA,
            pltpu.SemaphoreType.DMA,
            pltpu.SemaphoreType.DMA,
        ],
        compiler_params=pltpu.CompilerParams(use_tc_tiling_on_sc=False),
    )
    def k(idx_hbm, table_hbm, out_hbm, idx_v, rows_v, g0, g1, s0, s1, i0, i1):
        gsem = (g0, g1)
        ssem = (s0, s1)
        isem = (i0, i1)
        wid = lax.axis_index("s") * NC + lax.axis_index("c")
        bat0 = wid * bat_per_w

        def idx_copy(g, slot):
            return pltpu.make_async_copy(
                idx_hbm.at[pl.ds(bat0 + g * BSTAGE, BSTAGE)],
                idx_v.at[slot],
                isem[slot],
            )

        def gather_copies(slot):
            out = []
            for i in range(BSTAGE):
                for c in range(len(SPLITS) - 1):
                    lo, hi = SPLITS[c], SPLITS[c + 1]
                    out.append(
                        pltpu.make_async_copy(
                            table_hbm.at[idx_v.at[slot, i, pl.ds(lo, hi - lo)]],
                            rows_v.at[slot, i, pl.ds(lo, hi - lo)],
                            gsem[slot],
                        )
                    )
            return out

        def store_copy(g, slot):
            return pltpu.make_async_copy(
                rows_v.at[slot],
                out_hbm.at[pl.ds(bat0 + g * BSTAGE, BSTAGE)],
                ssem[slot],
            )

        def stage(g, b, first=False, prefetch=True):
            nb = 1 - b
            for d in gather_copies(b):
                d.wait()
            store_copy(g, b).start()
            idx_copy(g + 1, nb).wait()
            if not first:
                store_copy(g - 1, nb).wait()
            for d in gather_copies(nb):
                d.start()
            if prefetch:
                idx_copy(g + 2, b).start()

        idx_copy(0, 0).start()
        idx_copy(0, 0).wait()
        for d in gather_copies(0):
            d.start()
        idx_copy(1, 1).start()

        stage(0, 0, first=True)
        stage(1, 1)

        def body(h, carry):
            for b in range(2):
                stage(2 * h + b, b)
            return carry

        lax.fori_loop(1, n_stage // 2 - 1, body, 0)

        stage(n_stage - 2, 0, prefetch=False)
        for d in gather_copies(1):
            d.wait()
        store_copy(n_stage - 1, 1).start()

        store_copy(n_stage - 2, 0).wait()
        store_copy(n_stage - 1, 1).wait()

    return k


def _format_kernel(B_, S, D, blk_per_w):
    mesh = plsc.VectorSubcoreMesh(core_axis_name="c", subcore_axis_name="s")
    DB = D // 8

    @functools.partial(
        pl.kernel,
        out_type=jax.ShapeDtypeStruct((S, DB, B_ // BB, 8, BB), jnp.float32),
        mesh=mesh,
        scratch_types=[
            pltpu.VMEM((2, BB, RPAD), jnp.float32),
            pltpu.VMEM((2, DB, 8, BB), jnp.float32),
            pltpu.SemaphoreType.DMA,
            pltpu.SemaphoreType.DMA,
            pltpu.SemaphoreType.DMA,
            pltpu.SemaphoreType.DMA,
        ],
        compiler_params=pltpu.CompilerParams(
            use_tc_tiling_on_sc=False, needs_layout_passes=False
        ),
    )
    def k(x_hbm, out_hbm, rows_v, tile_v, g0, g1, s0, s1):
        gsem = (g0, g1)
        ssem = (s0, s1)
        wid = lax.axis_index("s") * NC + lax.axis_index("c")
        bblk0 = wid * blk_per_w

        lane = lax.iota(jnp.int32, L)
        row_base = [(lane + h * L) * RPAD for h in range(BB // L)]

        def load_copy(s, slot, bblk):
            return pltpu.make_async_copy(
                x_hbm.at[pl.ds((bblk0 + bblk) * BB, BB), s],
                rows_v.at[slot, :, pl.ds(0, D)],
                gsem[slot],
            )

        def store_copy(s, slot, bblk):
            return pltpu.make_async_copy(
                tile_v.at[slot],
                out_hbm.at[s, :, bblk0 + bblk],
                ssem[slot],
            )

        def full_transpose(slot):
            @plsc.parallel_loop(0, D, step=1, unroll=16)
            def dbody(d):
                col = jnp.full((L,), 0, jnp.int32) + d
                for h in range(BB // L):
                    v = plsc.load_gather(
                        rows_v.at[slot], [lane + h * L, col]
                    )
                    tile_v[slot, d // 8, d % 8, pl.ds(h * L, L)] = v

        for bblk in range(blk_per_w):
            load_copy(0, 0, bblk).start()

            def sbody(h2, carry):
                for b in range(2):
                    s = 2 * h2 + b
                    nb = 1 - b
                    @pl.when(s + 1 < S)
                    def _():
                        load_copy(s + 1, nb, bblk).start()

                    load_copy(s, b, bblk).wait()

                    @pl.when(s >= 2)
                    def _():
                        store_copy(s - 2, b, bblk).wait()

                    full_transpose(b)
                    store_copy(s, b, bblk).start()
                return carry

            lax.fori_loop(0, S // 2, sbody, 0)

            store_copy(S - 2, 0, bblk).wait()
            store_copy(S - 1, 1, bblk).wait()

    return k


def kernel(intent_ids, table):
    B_, S = intent_ids.shape
    V, D = table.shape
    assert B_ % (NW * BB) == 0 and B_ % (NW * BSTAGE) == 0
    assert D % 8 == 0 and S % 2 == 0
    bat_per_w = B_ // NW
    n_stage = bat_per_w // BSTAGE
    blk_per_w = B_ // (NW * BB)

    x = _gather_kernel(B_, S, V, D, bat_per_w, n_stage)(intent_ids, table)
    out5 = _format_kernel(B_, S, D, blk_per_w)(x)
    return out5.transpose(2, 4, 0, 1, 3).reshape(B_, S, D)


# speedup vs baseline: 1.0878x; 1.0005x over previous
import functools
import jax
import jax.numpy as jnp
from jax import lax
from jax.experimental import pallas as pl
from jax.experimental.pallas import tpu as pltpu
from jax.experimental.pallas import tpu_sc as plsc

NC = 2
NS = 16
NW = NC * NS
BSTAGE = 4
SPLITS = (0, 104, 200)

BB = 128
L = 16
RPAD = 65


def _gather_kernel(B_, S, V, D, bat_per_w, n_stage):
    mesh = plsc.VectorSubcoreMesh(core_axis_name="c", subcore_axis_name="s")

    @functools.partial(
        pl.kernel,
        out_type=jax.ShapeDtypeStruct((B_, S, D), jnp.float32),
        mesh=mesh,
        scratch_types=[
            pltpu.VMEM((2, BSTAGE, S), jnp.int32),
            pltpu.VMEM((2, BSTAGE, S, D), jnp.float32),
            pltpu.SemaphoreType.DMA,
            pltpu.SemaphoreType.DMA,
            pltpu.SemaphoreType.DMA,
            pltpu.SemaphoreType.DMA,
            pltpu.SemaphoreType.DMA,
            pltpu.SemaphoreType.DMA,
        ],
        compiler_params=pltpu.CompilerParams(use_tc_tiling_on_sc=False),
    )
    def k(idx_hbm, table_hbm, out_hbm, idx_v, rows_v, g0, g1, s0, s1, i0, i1):
        gsem = (g0, g1)
        ssem = (s0, s1)
        isem = (i0, i1)
        wid = lax.axis_index("s") * NC + lax.axis_index("c")
        bat0 = wid * bat_per_w

        def idx_copy(g, slot):
            return pltpu.make_async_copy(
                idx_hbm.at[pl.ds(bat0 + g * BSTAGE, BSTAGE)],
                idx_v.at[slot],
                isem[slot],
            )

        def gather_copies(slot):
            out = []
            for i in range(BSTAGE):
                for c in range(len(SPLITS) - 1):
                    lo, hi = SPLITS[c], SPLITS[c + 1]
                    out.append(
                        pltpu.make_async_copy(
                            table_hbm.at[idx_v.at[slot, i, pl.ds(lo, hi - lo)]],
                            rows_v.at[slot, i, pl.ds(lo, hi - lo)],
                            gsem[slot],
                        )
                    )
            return out

        def store_copy(g, slot):
            return pltpu.make_async_copy(
                rows_v.at[slot],
                out_hbm.at[pl.ds(bat0 + g * BSTAGE, BSTAGE)],
                ssem[slot],
            )

        def stage(g, b, first=False, prefetch=True):
            nb = 1 - b
            idx_copy(g + 1, nb).wait()
            if not first:
                store_copy(g - 1, nb).wait()
            for d in gather_copies(nb):
                d.start()
            for d in gather_copies(b):
                d.wait()
            store_copy(g, b).start()
            if prefetch:
                idx_copy(g + 2, b).start()

        idx_copy(0, 0).start()
        idx_copy(0, 0).wait()
        for d in gather_copies(0):
            d.start()
        idx_copy(1, 1).start()

        stage(0, 0, first=True)
        stage(1, 1)

        def body(h, carry):
            for b in range(2):
                stage(2 * h + b, b)
            return carry

        lax.fori_loop(1, n_stage // 2 - 1, body, 0)

        stage(n_stage - 2, 0, prefetch=False)
        for d in gather_copies(1):
            d.wait()
        store_copy(n_stage - 1, 1).start()

        store_copy(n_stage - 2, 0).wait()
        store_copy(n_stage - 1, 1).wait()

    return k


def _format_kernel(B_, S, D, blk_per_w):
    mesh = plsc.VectorSubcoreMesh(core_axis_name="c", subcore_axis_name="s")
    DB = D // 8

    @functools.partial(
        pl.kernel,
        out_type=jax.ShapeDtypeStruct((S, DB, B_ // BB, 8, BB), jnp.float32),
        mesh=mesh,
        scratch_types=[
            pltpu.VMEM((2, BB, RPAD), jnp.float32),
            pltpu.VMEM((2, DB, 8, BB), jnp.float32),
            pltpu.SemaphoreType.DMA,
            pltpu.SemaphoreType.DMA,
            pltpu.SemaphoreType.DMA,
            pltpu.SemaphoreType.DMA,
        ],
        compiler_params=pltpu.CompilerParams(
            use_tc_tiling_on_sc=False, needs_layout_passes=False
        ),
    )
    def k(x_hbm, out_hbm, rows_v, tile_v, g0, g1, s0, s1):
        gsem = (g0, g1)
        ssem = (s0, s1)
        wid = lax.axis_index("s") * NC + lax.axis_index("c")
        bblk0 = wid * blk_per_w

        lane = lax.iota(jnp.int32, L)
        row_base = [(lane + h * L) * RPAD for h in range(BB // L)]

        def load_copy(s, slot, bblk):
            return pltpu.make_async_copy(
                x_hbm.at[pl.ds((bblk0 + bblk) * BB, BB), s],
                rows_v.at[slot, :, pl.ds(0, D)],
                gsem[slot],
            )

        def store_copy(s, slot, bblk):
            return pltpu.make_async_copy(
                tile_v.at[slot],
                out_hbm.at[s, :, bblk0 + bblk],
                ssem[slot],
            )

        def full_transpose(slot):
            @plsc.parallel_loop(0, D, step=1, unroll=16)
            def dbody(d):
                col = jnp.full((L,), 0, jnp.int32) + d
                for h in range(BB // L):
                    v = plsc.load_gather(
                        rows_v.at[slot], [lane + h * L, col]
                    )
                    tile_v[slot, d // 8, d % 8, pl.ds(h * L, L)] = v

        for bblk in range(blk_per_w):
            load_copy(0, 0, bblk).start()

            def sbody(h2, carry):
                for b in range(2):
                    s = 2 * h2 + b
                    nb = 1 - b
                    @pl.when(s + 1 < S)
                    def _():
                        load_copy(s + 1, nb, bblk).start()

                    load_copy(s, b, bblk).wait()

                    @pl.when(s >= 2)
                    def _():
                        store_copy(s - 2, b, bblk).wait()

                    full_transpose(b)
                    store_copy(s, b, bblk).start()
                return carry

            lax.fori_loop(0, S // 2, sbody, 0)

            store_copy(S - 2, 0, bblk).wait()
            store_copy(S - 1, 1, bblk).wait()

    return k


def kernel(intent_ids, table):
    B_, S = intent_ids.shape
    V, D = table.shape
    assert B_ % (NW * BB) == 0 and B_ % (NW * BSTAGE) == 0
    assert D % 8 == 0 and S % 2 == 0
    bat_per_w = B_ // NW
    n_stage = bat_per_w // BSTAGE
    blk_per_w = B_ // (NW * BB)

    x = _gather_kernel(B_, S, V, D, bat_per_w, n_stage)(intent_ids, table)
    out5 = _format_kernel(B_, S, D, blk_per_w)(x)
    return out5.transpose(2, 4, 0, 1, 3).reshape(B_, S, D)
